# Initial kernel scaffold; baseline (speedup 1.0000x reference)
import jax
import jax.numpy as jnp
from jax.experimental import pallas as pl


def kernel(input_ids, table, W_proj, b_proj):
    raise NotImplementedError("write your pallas kernel here")


# speedup vs baseline: 1.1446x; 1.1446x over previous
import functools

import jax
import jax.numpy as jnp
from jax import lax
from jax.experimental import pallas as pl
from jax.experimental.pallas import tpu as pltpu
from jax.experimental.pallas import tpu_sc as plsc

_CH = 128


@functools.lru_cache(maxsize=None)
def _build_gather(n_pad, vocab, d_feat):
    info = plsc.get_sparse_core_info()
    nc, ns = info.num_cores, info.num_subcores
    nw = nc * ns
    b_per_w = n_pad // nw
    nch = b_per_w // _CH
    mesh = plsc.VectorSubcoreMesh(core_axis_name="c", subcore_axis_name="s")

    @functools.partial(
        pl.kernel,
        mesh=mesh,
        out_type=jax.ShapeDtypeStruct((n_pad, d_feat), jnp.float32),
        scratch_types=[
            pltpu.VMEM((_CH,), jnp.int32),
            pltpu.VMEM((_CH, d_feat), jnp.float32),
            pltpu.SemaphoreType.DMA,
        ],
        compiler_params=pltpu.CompilerParams(use_tc_tiling_on_sc=False),
    )
    def gather_kernel(table_hbm, idx_hbm, out_hbm, idx_v, rows_v, sem):
        wid = lax.axis_index("s") * nc + lax.axis_index("c")
        base = wid * b_per_w

        def body(j, carry):
            gbase = base + j * _CH
            pltpu.sync_copy(idx_hbm.at[pl.ds(gbase, _CH)], idx_v)
            pltpu.async_copy(table_hbm.at[idx_v], rows_v, sem).wait()
            pltpu.sync_copy(rows_v, out_hbm.at[pl.ds(gbase, _CH)])
            return carry

        lax.fori_loop(0, nch, body, 0)

    return gather_kernel


def _proj_body(emb_ref, w_ref, b_ref, out_ref):
    x = jnp.dot(emb_ref[...], w_ref[...], preferred_element_type=jnp.float32)
    x = x + b_ref[...]
    out_ref[...] = 0.5 * x * (1.0 + lax.erf(x * 0.7071067811865476))


@functools.lru_cache(maxsize=None)
def _build_project(n_pad, d_feat, d_model, blk):
    return pl.pallas_call(
        _proj_body,
        grid=(n_pad // blk,),
        in_specs=[
            pl.BlockSpec((blk, d_feat), lambda i: (i, 0)),
            pl.BlockSpec((d_feat, d_model), lambda i: (0, 0)),
            pl.BlockSpec((1, d_model), lambda i: (0, 0)),
        ],
        out_specs=pl.BlockSpec((blk, d_model), lambda i: (i, 0)),
        out_shape=jax.ShapeDtypeStruct((n_pad, d_model), jnp.float32),
    )


def kernel(input_ids, table, W_proj, b_proj):
    b, l = input_ids.shape
    vocab, d_feat = table.shape
    d_model = W_proj.shape[1]
    n = b * l

    ids = input_ids.reshape(-1).astype(jnp.int32)
    align = 32 * _CH
    n_pad = ((n + align - 1) // align) * align
    if n_pad != n:
        ids = jnp.concatenate([ids, jnp.zeros((n_pad - n,), jnp.int32)])

    emb = _build_gather(n_pad, vocab, d_feat)(table, ids)

    blk = 4096
    while n_pad % blk != 0:
        blk //= 2
    out = _build_project(n_pad, d_feat, d_model, blk)(
        emb, W_proj, b_proj.reshape(1, d_model)
    )
    return out[:n].reshape(b, l, d_model)


# speedup vs baseline: 1.3446x; 1.1747x over previous
import functools

import jax
import jax.numpy as jnp
from jax import lax
from jax.experimental import pallas as pl
from jax.experimental.pallas import tpu as pltpu
from jax.experimental.pallas import tpu_sc as plsc

_CH = 128
_NB = 4


@functools.lru_cache(maxsize=None)
def _build_gather(n_pad, vocab, d_feat):
    info = plsc.get_sparse_core_info()
    nc, ns = info.num_cores, info.num_subcores
    nw = nc * ns
    b_per_w = n_pad // nw
    nch = b_per_w // _CH
    ngroups = nch // _NB
    assert nch % _NB == 0 and ngroups % 2 == 0
    mesh = plsc.VectorSubcoreMesh(core_axis_name="c", subcore_axis_name="s")

    scratch = [pltpu.VMEM((b_per_w,), jnp.int32)]
    scratch += [pltpu.VMEM((_CH, d_feat), jnp.float32) for _ in range(2 * _NB)]
    scratch += [pltpu.SemaphoreType.DMA] * 4

    @functools.partial(
        pl.kernel,
        mesh=mesh,
        out_type=jax.ShapeDtypeStruct((n_pad, d_feat), jnp.float32),
        scratch_types=scratch,
        compiler_params=pltpu.CompilerParams(use_tc_tiling_on_sc=False),
    )
    def gather_kernel(table_hbm, idx_hbm, out_hbm, idx_v, *rest):
        bufs = rest[: 2 * _NB]
        gs0, gs1, ws0, ws1 = rest[2 * _NB :]
        set0, set1 = bufs[:_NB], bufs[_NB:]
        wid = lax.axis_index("s") * nc + lax.axis_index("c")
        base = wid * b_per_w
        pltpu.sync_copy(idx_hbm.at[pl.ds(base, b_per_w)], idx_v)

        def fire_gathers(grp, bufset, sem):
            for b in range(_NB):
                off = (grp * _NB + b) * _CH
                pltpu.async_copy(
                    table_hbm.at[idx_v.at[pl.ds(off, _CH)]], bufset[b], sem
                )

        def drain_gathers(grp, bufset, sem):
            for b in range(_NB):
                off = (grp * _NB + b) * _CH
                pltpu.make_async_copy(
                    table_hbm.at[idx_v.at[pl.ds(off, _CH)]], bufset[b], sem
                ).wait()

        def write_group(grp, bufset, sem):
            for b in range(_NB):
                off = base + (grp * _NB + b) * _CH
                pltpu.async_copy(bufset[b], out_hbm.at[pl.ds(off, _CH)], sem)
            for b in range(_NB):
                off = base + (grp * _NB + b) * _CH
                pltpu.make_async_copy(
                    bufset[b], out_hbm.at[pl.ds(off, _CH)], sem
                ).wait()

        fire_gathers(0, set0, gs0)

        def body(g2, carry):
            i0 = 2 * g2
            fire_gathers(i0 + 1, set1, gs1)
            drain_gathers(i0, set0, gs0)
            write_group(i0, set0, ws0)
            fire_gathers(lax.rem(i0 + 2, ngroups), set0, gs0)
            drain_gathers(i0 + 1, set1, gs1)
            write_group(i0 + 1, set1, ws1)
            return carry

        lax.fori_loop(0, ngroups // 2, body, 0)
        drain_gathers(0, set0, gs0)

    return gather_kernel


def _proj_body(emb_ref, w_ref, b_ref, out_ref):
    x = jnp.dot(emb_ref[...], w_ref[...], preferred_element_type=jnp.float32)
    x = x + b_ref[...]
    out_ref[...] = 0.5 * x * (1.0 + lax.erf(x * 0.7071067811865476))


@functools.lru_cache(maxsize=None)
def _build_project(n_pad, d_feat, d_model, blk):
    return pl.pallas_call(
        _proj_body,
        grid=(n_pad // blk,),
        in_specs=[
            pl.BlockSpec((blk, d_feat), lambda i: (i, 0)),
            pl.BlockSpec((d_feat, d_model), lambda i: (0, 0)),
            pl.BlockSpec((1, d_model), lambda i: (0, 0)),
        ],
        out_specs=pl.BlockSpec((blk, d_model), lambda i: (i, 0)),
        out_shape=jax.ShapeDtypeStruct((n_pad, d_model), jnp.float32),
    )


def kernel(input_ids, table, W_proj, b_proj):
    b, l = input_ids.shape
    vocab, d_feat = table.shape
    d_model = W_proj.shape[1]
    n = b * l

    ids = input_ids.reshape(-1).astype(jnp.int32)
    align = 32 * _CH * _NB * 2
    n_pad = ((n + align - 1) // align) * align
    if n_pad != n:
        ids = jnp.concatenate([ids, jnp.zeros((n_pad - n,), jnp.int32)])

    emb = _build_gather(n_pad, vocab, d_feat)(table, ids)

    blk = 4096
    while n_pad % blk != 0:
        blk //= 2
    out = _build_project(n_pad, d_feat, d_model, blk)(
        emb, W_proj, b_proj.reshape(1, d_model)
    )
    return out[:n].reshape(b, l, d_model)
